# baseline (speedup 1.0000x reference)
import functools

import jax
import jax.numpy as jnp
from jax.experimental import pallas as pl
from jax.experimental.pallas import tpu as pltpu


def _pick_block(n, target):
    best = None
    for b in range(8, min(n, target) + 1, 8):
        if n % b == 0:
            best = b
    return best if best is not None else n


def _dot(a, b):
    return jax.lax.dot_general(
        a, b, (((1,), (0,)), ((), ())),
        preferred_element_type=jnp.float32,
        precision=jax.lax.Precision.DEFAULT)


def _support_kernel(x_ref, w_ref, o_ref):
    o_ref[...] = jax.lax.dot_general(
        x_ref[...], w_ref[...], (((1,), (0,)), ((), ())),
        preferred_element_type=jnp.float32,
        precision=jax.lax.Precision.HIGHEST).astype(jnp.bfloat16)


def _fused_kernel(adj_ref, sup1_ref, b1_ref, w2_ref, b2_ref,
                  out_ref, sup2_ref, *, bi):
    l = pl.program_id(0)
    i = pl.program_id(1)
    adj_bf = adj_ref[...].astype(jnp.bfloat16)

    @pl.when(l == 0)
    def _pass_b():
        acc = _dot(adj_bf, sup1_ref[...])
        h = jnp.maximum(acc + b1_ref[...], 0.0)
        sup2_ref[pl.ds(i * bi, bi), :] = _dot(
            h.astype(jnp.bfloat16), w2_ref[...]).astype(jnp.bfloat16)

    @pl.when(l == 1)
    def _pass_c():
        out_ref[...] = _dot(adj_bf, sup2_ref[...]) + b2_ref[...]


def kernel(x, adj, W1, b1, W2, b2):
    n, d = x.shape
    bi = _pick_block(n, 200)
    bs = _pick_block(n, 1000)

    b1r = b1.reshape(1, d)
    b2r = b2.reshape(1, d)
    w2_bf = W2.astype(jnp.bfloat16)

    support1 = pl.pallas_call(
        _support_kernel,
        grid=(n // bs,),
        in_specs=[
            pl.BlockSpec((bs, d), lambda i: (i, 0)),
            pl.BlockSpec((d, d), lambda i: (0, 0)),
        ],
        out_specs=pl.BlockSpec((bs, d), lambda i: (i, 0)),
        out_shape=jax.ShapeDtypeStruct((n, d), jnp.bfloat16),
        compiler_params=pltpu.CompilerParams(
            dimension_semantics=("arbitrary",)),
    )(x, W1)

    out = pl.pallas_call(
        functools.partial(_fused_kernel, bi=bi),
        grid=(2, n // bi),
        in_specs=[
            pl.BlockSpec((bi, n), lambda l, i: (i, 0)),
            pl.BlockSpec((n, d), lambda l, i: (0, 0)),
            pl.BlockSpec((1, d), lambda l, i: (0, 0)),
            pl.BlockSpec((d, d), lambda l, i: (0, 0)),
            pl.BlockSpec((1, d), lambda l, i: (0, 0)),
        ],
        out_specs=pl.BlockSpec(
            (bi, d), lambda l, i: (jnp.where(l == 0, 0, i), 0)),
        out_shape=jax.ShapeDtypeStruct((n, d), jnp.float32),
        scratch_shapes=[
            pltpu.VMEM((n, d), jnp.bfloat16),
        ],
        compiler_params=pltpu.CompilerParams(
            dimension_semantics=("arbitrary", "arbitrary")),
    )(adj, support1, b1r, w2_bf, b2r)

    return out


# speedup vs baseline: 1.1418x; 1.1418x over previous
import jax
import jax.numpy as jnp
from jax.experimental import pallas as pl
from jax.experimental.pallas import tpu as pltpu


def _pick_block(n, target):
    best = None
    for b in range(8, min(n, target) + 1, 8):
        if n % b == 0:
            best = b
    return best if best is not None else n


def _dot(a, b):
    return jax.lax.dot_general(
        a, b, (((1,), (0,)), ((), ())),
        preferred_element_type=jnp.float32,
        precision=jax.lax.Precision.DEFAULT)


def _support_kernel(x_ref, w_ref, o_ref):
    o_ref[...] = jax.lax.dot_general(
        x_ref[...], w_ref[...], (((1,), (0,)), ((), ())),
        preferred_element_type=jnp.float32,
        precision=jax.lax.Precision.HIGHEST).astype(jnp.bfloat16)


def _layer1_kernel(adj_ref, sup_ref, b_ref, w2_ref, out_ref):
    acc = _dot(adj_ref[...].astype(jnp.bfloat16), sup_ref[...])
    h = jnp.maximum(acc + b_ref[...], 0.0)
    out_ref[...] = _dot(
        h.astype(jnp.bfloat16), w2_ref[...]).astype(jnp.bfloat16)


def _layer2_kernel(adj_ref, sup_ref, b_ref, out_ref):
    acc = _dot(adj_ref[...].astype(jnp.bfloat16), sup_ref[...])
    out_ref[...] = acc + b_ref[...]


def kernel(x, adj, W1, b1, W2, b2):
    n, d = x.shape
    bi = _pick_block(n, 400)

    b1r = b1.reshape(1, d)
    b2r = b2.reshape(1, d)
    w2_bf = W2.astype(jnp.bfloat16)

    support1 = pl.pallas_call(
        _support_kernel,
        grid=(n // bi,),
        in_specs=[
            pl.BlockSpec((bi, d), lambda i: (i, 0)),
            pl.BlockSpec((d, d), lambda i: (0, 0)),
        ],
        out_specs=pl.BlockSpec((bi, d), lambda i: (i, 0)),
        out_shape=jax.ShapeDtypeStruct((n, d), jnp.bfloat16),
        compiler_params=pltpu.CompilerParams(
            dimension_semantics=("arbitrary",)),
    )(x, W1)

    grid = (n // bi,)

    support2 = pl.pallas_call(
        _layer1_kernel,
        grid=grid,
        in_specs=[
            pl.BlockSpec((bi, n), lambda i: (i, 0)),
            pl.BlockSpec((n, d), lambda i: (0, 0)),
            pl.BlockSpec((1, d), lambda i: (0, 0)),
            pl.BlockSpec((d, d), lambda i: (0, 0)),
        ],
        out_specs=pl.BlockSpec((bi, d), lambda i: (i, 0)),
        out_shape=jax.ShapeDtypeStruct((n, d), jnp.bfloat16),
        compiler_params=pltpu.CompilerParams(
            dimension_semantics=("parallel",)),
    )(adj, support1, b1r, w2_bf)

    out = pl.pallas_call(
        _layer2_kernel,
        grid=grid,
        in_specs=[
            pl.BlockSpec((bi, n), lambda i: (i, 0)),
            pl.BlockSpec((n, d), lambda i: (0, 0)),
            pl.BlockSpec((1, d), lambda i: (0, 0)),
        ],
        out_specs=pl.BlockSpec((bi, d), lambda i: (i, 0)),
        out_shape=jax.ShapeDtypeStruct((n, d), jnp.float32),
        compiler_params=pltpu.CompilerParams(
            dimension_semantics=("parallel",)),
    )(adj, support2, b2r)

    return out
